# baseline (speedup 1.0000x reference)
import functools

import jax
import jax.numpy as jnp
from jax import lax
from jax.experimental import pallas as pl
from jax.experimental.pallas import tpu as pltpu
from jax.experimental.pallas import tpu_sc as plsc

B_ROWS = 16384 * 50
DIM = 32
NUM_CORES = 2
NUM_SUBCORES = 16
NW = NUM_CORES * NUM_SUBCORES
ROWS_PER_W = B_ROWS // NW
CHUNK = 1024
NCHUNK = ROWS_PER_W // CHUNK

_mesh = plsc.VectorSubcoreMesh(core_axis_name="c", subcore_axis_name="s")


@functools.partial(
    pl.kernel,
    mesh=_mesh,
    compiler_params=pltpu.CompilerParams(use_tc_tiling_on_sc=False),
    out_type=jax.ShapeDtypeStruct((B_ROWS, DIM), jnp.float32),
    scratch_types=[
        pltpu.VMEM_SHARED((NUM_SUBCORES, ROWS_PER_W), jnp.int32),
        pltpu.SMEM((CHUNK,), jnp.int32),
        pltpu.SemaphoreType.DMA,
    ],
)
def _emb_lookup(idx_hbm, w_hbm, out_hbm, idx_sh, idx_s, sem):
    cid = lax.axis_index("c")
    sid = lax.axis_index("s")
    wid = sid * NUM_CORES + cid
    base = wid * ROWS_PER_W

    pltpu.sync_copy(idx_hbm.at[pl.ds(base, ROWS_PER_W)], idx_sh.at[sid])

    def chunk_body(c, carry):
        off = base + c * CHUNK
        pltpu.sync_copy(idx_sh.at[sid].at[pl.ds(c * CHUNK, CHUNK)], idx_s)

        def row_body(j, carry2):
            pltpu.async_copy(w_hbm.at[pl.ds(idx_s[j], 1)],
                             out_hbm.at[pl.ds(off + j, 1)], sem)
            return carry2

        lax.fori_loop(0, CHUNK, row_body, 0, unroll=8)
        pltpu.make_async_copy(w_hbm.at[pl.ds(0, CHUNK)],
                              out_hbm.at[pl.ds(off, CHUNK)], sem).wait()
        return carry

    lax.fori_loop(0, NCHUNK, chunk_body, 0)


def kernel(x, w):
    flat = x.reshape(-1).astype(jnp.int32)
    out = _emb_lookup(flat, w)
    return out.reshape(x.shape + (DIM,))


# speedup vs baseline: 1.9924x; 1.9924x over previous
import functools

import jax
import jax.numpy as jnp
from jax import lax
from jax.experimental import pallas as pl
from jax.experimental.pallas import tpu as pltpu
from jax.experimental.pallas import tpu_sc as plsc

B_ROWS = 16384 * 50
DIM = 32
NUM_CORES = 2
NUM_SUBCORES = 16
NW = NUM_CORES * NUM_SUBCORES
ROWS_PER_W = B_ROWS // NW

S_ROWS = 19200
CHUNK = 1200
NCHUNK = S_ROWS // CHUNK

D_ROWS = ROWS_PER_W - S_ROWS
BATCH = D_ROWS // (NCHUNK // 2)

_mesh = plsc.VectorSubcoreMesh(core_axis_name="c", subcore_axis_name="s")


@functools.partial(
    pl.kernel,
    mesh=_mesh,
    compiler_params=pltpu.CompilerParams(use_tc_tiling_on_sc=False),
    out_type=jax.ShapeDtypeStruct((B_ROWS, DIM), jnp.float32),
    scratch_types=[
        pltpu.VMEM((S_ROWS,), jnp.int32),
        pltpu.VMEM((CHUNK, DIM), jnp.float32),
        pltpu.VMEM((CHUNK, DIM), jnp.float32),
        pltpu.VMEM_SHARED((NUM_SUBCORES, D_ROWS), jnp.int32),
        pltpu.SMEM((BATCH,), jnp.int32),
        pltpu.SemaphoreType.DMA,
        pltpu.SemaphoreType.DMA,
        pltpu.SemaphoreType.DMA,
    ],
)
def _emb_lookup(idx_hbm, w_hbm, out_hbm, idx_v, rows0, rows1, idx_dma_sh,
                idx_s, sem0, sem1, dsem):
    cid = lax.axis_index("c")
    sid = lax.axis_index("s")
    wid = sid * NUM_CORES + cid
    base = wid * ROWS_PER_W
    dbase = base + S_ROWS

    bufs = (rows0, rows1)
    sems = (sem0, sem1)

    def fire(i, b):
        @pl.when(i < NCHUNK)
        def _():
            pltpu.async_copy(w_hbm.at[idx_v.at[pl.ds(i * CHUNK, CHUNK)]],
                             bufs[b], sems[b])

    def drain(b):
        pltpu.make_async_copy(w_hbm.at[idx_v.at[pl.ds(0, CHUNK)]],
                              bufs[b], sems[b]).wait()

    pltpu.sync_copy(idx_hbm.at[pl.ds(base, S_ROWS)], idx_v)
    pltpu.sync_copy(idx_hbm.at[pl.ds(dbase, D_ROWS)], idx_dma_sh.at[sid])
    fire(0, 0)

    def body(g, carry):
        i0 = g * 2
        fire(i0 + 1, 1)
        doff = dbase + g * BATCH
        pltpu.sync_copy(idx_dma_sh.at[sid].at[pl.ds(g * BATCH, BATCH)],
                        idx_s)

        def row_body(j, carry2):
            pltpu.async_copy(w_hbm.at[pl.ds(idx_s[j], 1)],
                             out_hbm.at[pl.ds(doff + j, 1)], dsem)
            return carry2

        lax.fori_loop(0, BATCH, row_body, 0)
        drain(0)
        pltpu.sync_copy(rows0, out_hbm.at[pl.ds(base + i0 * CHUNK, CHUNK)])
        fire(i0 + 2, 0)
        drain(1)
        pltpu.sync_copy(rows1, out_hbm.at[pl.ds(base + (i0 + 1) * CHUNK, CHUNK)])
        return carry

    lax.fori_loop(0, NCHUNK // 2, body, 0)

    def dma_drain(g, carry):
        pltpu.make_async_copy(w_hbm.at[pl.ds(0, BATCH)],
                              out_hbm.at[pl.ds(dbase, BATCH)], dsem).wait()
        return carry

    lax.fori_loop(0, NCHUNK // 2, dma_drain, 0)


def kernel(x, w):
    flat = x.reshape(-1).astype(jnp.int32)
    out = _emb_lookup(flat, w)
    return out.reshape(x.shape + (DIM,))


# speedup vs baseline: 2.8589x; 1.4349x over previous
import functools

import jax
import jax.numpy as jnp
from jax import lax
from jax.experimental import pallas as pl
from jax.experimental.pallas import tpu as pltpu
from jax.experimental.pallas import tpu_sc as plsc

B_ROWS = 16384 * 50
DIM = 32
NUM_CORES = 2
NUM_SUBCORES = 16
NW = NUM_CORES * NUM_SUBCORES
ROWS_PER_W = B_ROWS // NW
CHUNK = 1280
NCHUNK = ROWS_PER_W // CHUNK

_mesh = plsc.VectorSubcoreMesh(core_axis_name="c", subcore_axis_name="s")


@functools.partial(
    pl.kernel,
    mesh=_mesh,
    compiler_params=pltpu.CompilerParams(use_tc_tiling_on_sc=False),
    out_type=jax.ShapeDtypeStruct((B_ROWS, DIM), jnp.float32),
    scratch_types=[
        pltpu.VMEM((ROWS_PER_W,), jnp.int32),
        pltpu.VMEM((CHUNK, DIM), jnp.float32),
        pltpu.VMEM((CHUNK, DIM), jnp.float32),
        pltpu.SemaphoreType.DMA,
        pltpu.SemaphoreType.DMA,
    ],
)
def _emb_lookup(idx_hbm, w_hbm, out_hbm, idx_v, rows0, rows1, sem0, sem1):
    wid = lax.axis_index("s") * NUM_CORES + lax.axis_index("c")
    base = wid * ROWS_PER_W

    bufs = (rows0, rows1)
    sems = (sem0, sem1)

    def fire(i, b):
        @pl.when(i < NCHUNK)
        def _():
            def sub(k, c2):
                vec = idx_v[pl.ds(i * CHUNK + k * 16, 16)]
                pltpu.async_copy(w_hbm.at[vec],
                                 bufs[b].at[pl.ds(k * 16, 16)], sems[b])
                return c2
            lax.fori_loop(0, CHUNK // 16, sub, 0)

    def drain(b):
        pltpu.make_async_copy(w_hbm.at[idx_v.at[pl.ds(0, CHUNK)]],
                              bufs[b], sems[b]).wait()

    pltpu.sync_copy(idx_hbm.at[pl.ds(base, ROWS_PER_W)], idx_v)
    fire(0, 0)

    def body(g, carry):
        i0 = g * 2
        fire(i0 + 1, 1)
        drain(0)
        pltpu.sync_copy(rows0, out_hbm.at[pl.ds(base + i0 * CHUNK, CHUNK)])
        fire(i0 + 2, 0)
        drain(1)
        pltpu.sync_copy(rows1, out_hbm.at[pl.ds(base + (i0 + 1) * CHUNK, CHUNK)])
        return carry

    lax.fori_loop(0, NCHUNK // 2, body, 0)


def kernel(x, w):
    flat = x.reshape(-1).astype(jnp.int32)
    out = _emb_lookup(flat, w)
    return out.reshape(x.shape + (DIM,))
